# baseline (speedup 1.0000x reference)
import functools

import jax
import jax.numpy as jnp
from jax.experimental import pallas as pl
from jax.experimental.pallas import tpu as pltpu

_LOG2PI = 1.8378770664093453
_K = 16


def _tc_body(w_ref, mus_ref, lvs_ref, z_ref, out_ref, *, d_const):
    w = w_ref[0, :]
    m = jnp.max(w)
    lw = w - (m + jnp.log(jnp.sum(jnp.exp(w - m))))
    lv = lvs_ref[...]
    mu = mus_ref[...]
    gamma = -0.5 * jnp.exp(-lv)
    a = lw[:, None] - 0.5 * lv
    A = jnp.max(a, axis=0, keepdims=True)
    alpha = (a - A) + gamma * mu * mu
    beta = -2.0 * gamma * mu
    z = z_ref[...]
    z2 = z * z
    s = jnp.zeros_like(z)
    for k in range(_K):
        t = alpha[k][None, :] + beta[k][None, :] * z + gamma[k][None, :] * z2
        s = s + jnp.exp(t)
    out_ref[...] = (A + d_const) + jnp.log(s)


def kernel(z, mus, log_vars, w):
    B, L = z.shape
    d_const = -0.5 * B * _LOG2PI
    z2d = z.reshape(B // 2, 2 * L)
    mus2 = jnp.concatenate([mus, mus], axis=1)
    lvs2 = jnp.concatenate([log_vars, log_vars], axis=1)
    wf = w.reshape(1, _K)
    R = 512
    grid = (z2d.shape[0] // R,)
    out = pl.pallas_call(
        functools.partial(_tc_body, d_const=d_const),
        grid=grid,
        in_specs=[
            pl.BlockSpec((1, _K), lambda i: (0, 0)),
            pl.BlockSpec((_K, 2 * L), lambda i: (0, 0)),
            pl.BlockSpec((_K, 2 * L), lambda i: (0, 0)),
            pl.BlockSpec((R, 2 * L), lambda i: (i, 0)),
        ],
        out_specs=pl.BlockSpec((R, 2 * L), lambda i: (i, 0)),
        out_shape=jax.ShapeDtypeStruct(z2d.shape, jnp.float32),
    )(wf, mus2, lvs2, z2d)
    return out.reshape(B, L)
